# Initial kernel scaffold; baseline (speedup 1.0000x reference)
import jax
import jax.numpy as jnp
from jax.experimental import pallas as pl


def kernel(x, edge_index, edge_weight, W_z, b_z, Wl_z, bl_z, W_r, b_r, Wl_r, bl_r, W_h, b_h, Wl_h, bl_h, att, W_out, b_out):
    raise NotImplementedError("write your pallas kernel here")


# speedup vs baseline: 304.0239x; 304.0239x over previous
import functools

import jax
import jax.numpy as jnp
from jax import lax
from jax.experimental import pallas as pl
from jax.experimental.pallas import tpu as pltpu
from jax.experimental.pallas import tpu_sc as plsc

N = 50000
E = 800000
F_IN = 2
PERIODS = 12
HID = 32
FT = F_IN * PERIODS

NC = 2
NT = 16
NW = NC * NT
CHUNK = 128
CH = 196
EPT = CH * CHUNK
E_PAD = NW * EPT
ROWS_PER_TILE = 3136
N_PAD = NT * ROWS_PER_TILE

K = 7
SG = 28
NSG = CH // SG
ZR = 448
KD = 14
NSD = CH // KD

CB = 3584

_SC_MESH = plsc.VectorSubcoreMesh(core_axis_name="c", subcore_axis_name="s")


@functools.partial(
    pl.kernel,
    out_type=jax.ShapeDtypeStruct((NC * N_PAD,), jnp.float32),
    mesh=_SC_MESH,
    compiler_params=pltpu.CompilerParams(use_tc_tiling_on_sc=False),
    scratch_types=[
        pltpu.VMEM((CH, CHUNK), jnp.int32),
        pltpu.VMEM((CHUNK,), jnp.float32),
        pltpu.VMEM((ROWS_PER_TILE,), jnp.float32),
        pltpu.VMEM_SHARED((N_PAD,), jnp.float32),
        pltpu.SemaphoreType.DMA,
    ],
)
def _sc_deg(dst_hbm, zeros_hbm, deg_hbm, idx_v, ones_v, zero_v, deg_sh, sem):
    c = lax.axis_index("c")
    s = lax.axis_index("s")
    w = c * NT + s
    r0 = s * ROWS_PER_TILE
    pltpu.sync_copy(dst_hbm.at[w], idx_v)
    for i in range(CHUNK // 16):
        ones_v[pl.ds(i * 16, 16)] = jnp.full((16,), 1.0, jnp.float32)
    pltpu.sync_copy(zeros_hbm.at[pl.ds(0, ROWS_PER_TILE)], zero_v)
    pltpu.sync_copy(zero_v, deg_sh.at[pl.ds(r0, ROWS_PER_TILE)])
    plsc.subcore_barrier()

    def group(g, carry):
        base = g * KD
        descs = [
            pltpu.async_copy(ones_v, deg_sh.at[idx_v.at[base + j]], sem, add=True)
            for j in range(KD)
        ]
        for d in descs:
            d.wait()
        return carry

    lax.fori_loop(0, NSD, group, 0)
    plsc.subcore_barrier()
    pltpu.sync_copy(deg_sh.at[pl.ds(r0, ROWS_PER_TILE)], zero_v)
    pltpu.sync_copy(zero_v, deg_hbm.at[pl.ds(c * N_PAD + r0, ROWS_PER_TILE)])


@functools.partial(
    pl.kernel,
    out_type=jax.ShapeDtypeStruct((NC, N_PAD, FT), jnp.float32),
    mesh=_SC_MESH,
    compiler_params=pltpu.CompilerParams(use_tc_tiling_on_sc=False),
    scratch_types=[
        pltpu.VMEM((SG, CHUNK), jnp.int32),
        pltpu.VMEM((SG, CHUNK), jnp.int32),
        pltpu.VMEM((K, CHUNK, FT), jnp.float32),
        pltpu.VMEM((ZR, FT), jnp.float32),
        pltpu.VMEM_SHARED((N_PAD, FT), jnp.float32),
        pltpu.SemaphoreType.DMA,
        pltpu.SemaphoreType.DMA,
    ],
)
def _sc_scatter(src_hbm, dst_hbm, xs_hbm, zeros_hbm, y_hbm,
                src_v, dst_v, rows_v, zero_v, y_sh, sem_g, sem_s):
    c = lax.axis_index("c")
    s = lax.axis_index("s")
    w = c * NT + s
    r0 = s * ROWS_PER_TILE
    pltpu.sync_copy(zeros_hbm.at[pl.ds(0, ZR)], zero_v)
    for q in range(ROWS_PER_TILE // ZR):
        pltpu.sync_copy(zero_v, y_sh.at[pl.ds(r0 + q * ZR, ZR)])
    plsc.subcore_barrier()

    def supergroup(g, carry):
        pltpu.sync_copy(src_hbm.at[w, g], src_v)
        pltpu.sync_copy(dst_hbm.at[w, g], dst_v)

        def group(u, inner):
            base = u * K
            gets = [
                pltpu.async_copy(xs_hbm.at[src_v.at[base + j]], rows_v.at[j],
                                 sem_g)
                for j in range(K)
            ]
            for d in gets:
                d.wait()
            puts = [
                pltpu.async_copy(rows_v.at[j], y_sh.at[dst_v.at[base + j]],
                                 sem_s, add=True)
                for j in range(K)
            ]
            for d in puts:
                d.wait()
            return inner

        lax.fori_loop(0, SG // K, group, 0)
        return carry

    lax.fori_loop(0, NSG, supergroup, 0)
    plsc.subcore_barrier()
    for q in range(ROWS_PER_TILE // ZR):
        qr = q * ZR
        pltpu.sync_copy(y_sh.at[pl.ds(r0 + qr, ZR)], zero_v)
        pltpu.sync_copy(zero_v, y_hbm.at[c, pl.ds(r0 + qr, ZR)])


def _prescale_body(deg_ref, xt_ref, xst_ref, dis_ref):
    dis = lax.rsqrt(deg_ref[0:1, :] + deg_ref[1:2, :] + 1.0)
    dis_ref[...] = dis
    xst_ref[...] = xt_ref[...] * dis


def _prescale(deg2, xt):
    return pl.pallas_call(
        _prescale_body,
        grid=(N_PAD // CB,),
        in_specs=[
            pl.BlockSpec((NC, CB), lambda i: (0, i)),
            pl.BlockSpec((FT, CB), lambda i: (0, i)),
        ],
        out_specs=[
            pl.BlockSpec((FT, CB), lambda i: (0, i)),
            pl.BlockSpec((1, CB), lambda i: (0, i)),
        ],
        out_shape=[
            jax.ShapeDtypeStruct((FT, N_PAD), jnp.float32),
            jax.ShapeDtypeStruct((1, N_PAD), jnp.float32),
        ],
    )(deg2, xt)


def _gates_body(yp_ref, xst_ref, dis_ref, wzT_ref, wlzT_ref, bz_ref, blz_ref,
                whT_ref, wlhT_ref, bh_ref, blh_ref, att_ref, woutT_ref,
                bout_ref, out_ref):
    yf = dis_ref[...] * (yp_ref[0] + yp_ref[1] + xst_ref[...])
    wlz1 = wlzT_ref[...][:, :HID]
    wlh1 = wlhT_ref[...][:, :HID]
    azT = jnp.dot(wlz1, wzT_ref[...], preferred_element_type=jnp.float32)
    ahT = jnp.dot(wlh1, whT_ref[...], preferred_element_type=jnp.float32)
    czT = jnp.dot(wlz1, bz_ref[...], preferred_element_type=jnp.float32) + blz_ref[...]
    chT = jnp.dot(wlh1, bh_ref[...], preferred_element_type=jnp.float32) + blh_ref[...]
    a = att_ref[...]
    e = jnp.exp(a - jnp.max(a))
    p = e / jnp.sum(e)
    az0, az1 = azT[:, 0:1], azT[:, 1:2]
    ah0, ah1 = ahT[:, 0:1], ahT[:, 1:2]
    acc = jnp.zeros((HID, yf.shape[1]), jnp.float32)
    for t in range(PERIODS):
        y0 = yf[t:t + 1, :]
        y1 = yf[PERIODS + t:PERIODS + t + 1, :]
        z = jax.nn.sigmoid(az0 * y0 + az1 * y1 + czT)
        h = jnp.tanh(ah0 * y0 + ah1 * y1 + chT)
        acc = acc + p[t:t + 1, 0:1] * ((1.0 - z) * h)
    out = jnp.dot(woutT_ref[...], jnp.maximum(acc, 0.0),
                  preferred_element_type=jnp.float32)
    out_ref[...] = out + bout_ref[...]


def _gates(ypT, xst, dist, wzT, wlzT, bz, blz, whT, wlhT, bh, blh, attc,
           woutT, boutc):
    full = lambda i: (0, 0)
    return pl.pallas_call(
        _gates_body,
        grid=(N_PAD // CB,),
        in_specs=[
            pl.BlockSpec((NC, FT, CB), lambda i: (0, 0, i)),
            pl.BlockSpec((FT, CB), lambda i: (0, i)),
            pl.BlockSpec((1, CB), lambda i: (0, i)),
            pl.BlockSpec((HID, F_IN), full),
            pl.BlockSpec((HID, 2 * HID), full),
            pl.BlockSpec((HID, 1), full),
            pl.BlockSpec((HID, 1), full),
            pl.BlockSpec((HID, F_IN), full),
            pl.BlockSpec((HID, 2 * HID), full),
            pl.BlockSpec((HID, 1), full),
            pl.BlockSpec((HID, 1), full),
            pl.BlockSpec((PERIODS, 1), full),
            pl.BlockSpec((PERIODS, HID), full),
            pl.BlockSpec((PERIODS, 1), full),
        ],
        out_specs=pl.BlockSpec((PERIODS, CB), lambda i: (0, i)),
        out_shape=jax.ShapeDtypeStruct((PERIODS, N_PAD), jnp.float32),
    )(ypT, xst, dist, wzT, wlzT, bz, blz, whT, wlhT, bh, blh, attc, woutT,
      boutc)


def kernel(x, edge_index, edge_weight, W_z, b_z, Wl_z, bl_z, W_r, b_r, Wl_r,
           bl_r, W_h, b_h, Wl_h, bl_h, att, W_out, b_out):
    del edge_weight, W_r, b_r, Wl_r, bl_r
    src = edge_index[0]
    dst = edge_index[1]
    pad = jnp.full((E_PAD - E,), N, jnp.int32)
    src4 = jnp.concatenate([src, pad]).reshape(NW, NSG, SG, CHUNK)
    dst4 = jnp.concatenate([dst, pad]).reshape(NW, NSG, SG, CHUNK)
    dst3 = dst4.reshape(NW, CH, CHUNK)

    xflat = x.reshape(N, FT)
    xt = jnp.pad(xflat, ((0, N_PAD - N), (0, 0))).T

    zeros_row = jnp.zeros((N_PAD,), jnp.float32)
    zeros_tab = jnp.zeros((N_PAD, FT), jnp.float32)

    deg2 = _sc_deg(dst3, zeros_row).reshape(NC, N_PAD)
    xst, dist = _prescale(deg2, xt)
    xs = xst.T
    ypart = _sc_scatter(src4, dst4, xs, zeros_tab)
    ypT = jnp.transpose(ypart, (0, 2, 1))

    outT = _gates(
        ypT, xst, dist,
        W_z.T, Wl_z.T, b_z.reshape(HID, 1), bl_z.reshape(HID, 1),
        W_h.T, Wl_h.T, b_h.reshape(HID, 1), bl_h.reshape(HID, 1),
        att.reshape(PERIODS, 1), W_out.T, b_out.reshape(PERIODS, 1),
    )
    return outT.T[:N]


# speedup vs baseline: 323.2049x; 1.0631x over previous
import functools

import jax
import jax.numpy as jnp
from jax import lax
from jax.experimental import pallas as pl
from jax.experimental.pallas import tpu as pltpu
from jax.experimental.pallas import tpu_sc as plsc

N = 50000
E = 800000
F_IN = 2
PERIODS = 12
HID = 32
FT = F_IN * PERIODS

NC = 2
NT = 16
NW = NC * NT
CHUNK = 128
CH = 196
EPT = CH * CHUNK
E_PAD = NW * EPT
ROWS_PER_TILE = 3136
N_PAD = NT * ROWS_PER_TILE

K = 7
SG = 14
NSG = CH // SG
ZR = 784
KD = 14
NSD = CH // KD

CB = 3584

_SC_MESH = plsc.VectorSubcoreMesh(core_axis_name="c", subcore_axis_name="s")


@functools.partial(
    pl.kernel,
    out_type=jax.ShapeDtypeStruct((NC * N_PAD,), jnp.float32),
    mesh=_SC_MESH,
    compiler_params=pltpu.CompilerParams(use_tc_tiling_on_sc=False),
    scratch_types=[
        pltpu.VMEM((CH, CHUNK), jnp.int32),
        pltpu.VMEM((CHUNK,), jnp.float32),
        pltpu.VMEM((ROWS_PER_TILE,), jnp.float32),
        pltpu.VMEM_SHARED((N_PAD,), jnp.float32),
        pltpu.SemaphoreType.DMA,
    ],
)
def _sc_deg(dst_hbm, zeros_hbm, deg_hbm, idx_v, ones_v, zero_v, deg_sh, sem):
    c = lax.axis_index("c")
    s = lax.axis_index("s")
    w = c * NT + s
    r0 = s * ROWS_PER_TILE
    pltpu.sync_copy(dst_hbm.at[w], idx_v)
    for i in range(CHUNK // 16):
        ones_v[pl.ds(i * 16, 16)] = jnp.full((16,), 1.0, jnp.float32)
    pltpu.sync_copy(zeros_hbm.at[pl.ds(0, ROWS_PER_TILE)], zero_v)
    pltpu.sync_copy(zero_v, deg_sh.at[pl.ds(r0, ROWS_PER_TILE)])
    plsc.subcore_barrier()

    def group(g, carry):
        base = g * KD
        descs = [
            pltpu.async_copy(ones_v, deg_sh.at[idx_v.at[base + j]], sem, add=True)
            for j in range(KD)
        ]
        for d in descs:
            d.wait()
        return carry

    lax.fori_loop(0, NSD, group, 0)
    plsc.subcore_barrier()
    pltpu.sync_copy(deg_sh.at[pl.ds(r0, ROWS_PER_TILE)], zero_v)
    pltpu.sync_copy(zero_v, deg_hbm.at[pl.ds(c * N_PAD + r0, ROWS_PER_TILE)])


@functools.partial(
    pl.kernel,
    out_type=jax.ShapeDtypeStruct((NC, N_PAD, FT), jnp.float32),
    mesh=_SC_MESH,
    compiler_params=pltpu.CompilerParams(use_tc_tiling_on_sc=False),
    scratch_types=[
        pltpu.VMEM((2, SG, CHUNK), jnp.int32),
        pltpu.VMEM((2, SG, CHUNK), jnp.int32),
        pltpu.VMEM((2, K * CHUNK, FT), jnp.float32),
        pltpu.VMEM_SHARED((N_PAD, FT), jnp.float32),
        pltpu.SemaphoreType.DMA,
        pltpu.SemaphoreType.DMA,
        pltpu.SemaphoreType.DMA,
        pltpu.SemaphoreType.DMA,
    ],
)
def _sc_scatter(src_hbm, dst_hbm, xs_hbm, zeros_hbm, y_hbm,
                src_v, dst_v, rows_v, y_sh, sem_g0, sem_g1, sem_s0, sem_s1):
    c = lax.axis_index("c")
    s = lax.axis_index("s")
    w = c * NT + s
    r0 = s * ROWS_PER_TILE
    pltpu.sync_copy(zeros_hbm.at[pl.ds(0, ZR)], rows_v.at[0, pl.ds(0, ZR)])
    for q in range(ROWS_PER_TILE // ZR):
        pltpu.sync_copy(rows_v.at[0, pl.ds(0, ZR)],
                        y_sh.at[pl.ds(r0 + q * ZR, ZR)])
    plsc.subcore_barrier()

    pltpu.sync_copy(src_hbm.at[w, 0], src_v.at[0])
    pltpu.sync_copy(dst_hbm.at[w, 0], dst_v.at[0])

    def pair(p, carry):
        pb = p % 2
        gets0 = [
            pltpu.async_copy(xs_hbm.at[src_v.at[pb, j]],
                             rows_v.at[0, pl.ds(j * CHUNK, CHUNK)], sem_g0)
            for j in range(K)
        ]
        gets1 = [
            pltpu.async_copy(xs_hbm.at[src_v.at[pb, K + j]],
                             rows_v.at[1, pl.ds(j * CHUNK, CHUNK)], sem_g1)
            for j in range(K)
        ]
        for d in gets0:
            d.wait()
        puts0 = [
            pltpu.async_copy(rows_v.at[0, pl.ds(j * CHUNK, CHUNK)],
                             y_sh.at[dst_v.at[pb, j]], sem_s0, add=True)
            for j in range(K)
        ]
        for d in gets1:
            d.wait()
        puts1 = [
            pltpu.async_copy(rows_v.at[1, pl.ds(j * CHUNK, CHUNK)],
                             y_sh.at[dst_v.at[pb, K + j]], sem_s1, add=True)
            for j in range(K)
        ]

        @pl.when(p + 1 < NSG)
        def _():
            pltpu.sync_copy(src_hbm.at[w, p + 1], src_v.at[(p + 1) % 2])
            pltpu.sync_copy(dst_hbm.at[w, p + 1], dst_v.at[(p + 1) % 2])

        for d in puts0:
            d.wait()
        for d in puts1:
            d.wait()
        return carry

    lax.fori_loop(0, NSG, pair, 0)
    plsc.subcore_barrier()
    for q in range(ROWS_PER_TILE // ZR):
        qr = q * ZR
        pltpu.sync_copy(y_sh.at[pl.ds(r0 + qr, ZR)], rows_v.at[0, pl.ds(0, ZR)])
        pltpu.sync_copy(rows_v.at[0, pl.ds(0, ZR)],
                        y_hbm.at[c, pl.ds(r0 + qr, ZR)])


def _prescale_body(deg_ref, xt_ref, xst_ref, dis_ref):
    dis = lax.rsqrt(deg_ref[0:1, :] + deg_ref[1:2, :] + 1.0)
    dis_ref[...] = dis
    xst_ref[...] = xt_ref[...] * dis


def _prescale(deg2, xt):
    return pl.pallas_call(
        _prescale_body,
        grid=(N_PAD // CB,),
        in_specs=[
            pl.BlockSpec((NC, CB), lambda i: (0, i)),
            pl.BlockSpec((FT, CB), lambda i: (0, i)),
        ],
        out_specs=[
            pl.BlockSpec((FT, CB), lambda i: (0, i)),
            pl.BlockSpec((1, CB), lambda i: (0, i)),
        ],
        out_shape=[
            jax.ShapeDtypeStruct((FT, N_PAD), jnp.float32),
            jax.ShapeDtypeStruct((1, N_PAD), jnp.float32),
        ],
    )(deg2, xt)


def _gates_body(yp_ref, xst_ref, dis_ref, wzT_ref, wlzT_ref, bz_ref, blz_ref,
                whT_ref, wlhT_ref, bh_ref, blh_ref, att_ref, woutT_ref,
                bout_ref, out_ref):
    yf = dis_ref[...] * (yp_ref[0] + yp_ref[1] + xst_ref[...])
    wlz1 = wlzT_ref[...][:, :HID]
    wlh1 = wlhT_ref[...][:, :HID]
    azT = jnp.dot(wlz1, wzT_ref[...], preferred_element_type=jnp.float32)
    ahT = jnp.dot(wlh1, whT_ref[...], preferred_element_type=jnp.float32)
    czT = jnp.dot(wlz1, bz_ref[...], preferred_element_type=jnp.float32) + blz_ref[...]
    chT = jnp.dot(wlh1, bh_ref[...], preferred_element_type=jnp.float32) + blh_ref[...]
    a = att_ref[...]
    e = jnp.exp(a - jnp.max(a))
    p = e / jnp.sum(e)
    az0, az1 = azT[:, 0:1], azT[:, 1:2]
    ah0, ah1 = ahT[:, 0:1], ahT[:, 1:2]
    acc = jnp.zeros((HID, yf.shape[1]), jnp.float32)
    for t in range(PERIODS):
        y0 = yf[t:t + 1, :]
        y1 = yf[PERIODS + t:PERIODS + t + 1, :]
        z = jax.nn.sigmoid(az0 * y0 + az1 * y1 + czT)
        h = jnp.tanh(ah0 * y0 + ah1 * y1 + chT)
        acc = acc + p[t:t + 1, 0:1] * ((1.0 - z) * h)
    out = jnp.dot(woutT_ref[...], jnp.maximum(acc, 0.0),
                  preferred_element_type=jnp.float32)
    out_ref[...] = out + bout_ref[...]


def _gates(ypT, xst, dist, wzT, wlzT, bz, blz, whT, wlhT, bh, blh, attc,
           woutT, boutc):
    full = lambda i: (0, 0)
    return pl.pallas_call(
        _gates_body,
        grid=(N_PAD // CB,),
        in_specs=[
            pl.BlockSpec((NC, FT, CB), lambda i: (0, 0, i)),
            pl.BlockSpec((FT, CB), lambda i: (0, i)),
            pl.BlockSpec((1, CB), lambda i: (0, i)),
            pl.BlockSpec((HID, F_IN), full),
            pl.BlockSpec((HID, 2 * HID), full),
            pl.BlockSpec((HID, 1), full),
            pl.BlockSpec((HID, 1), full),
            pl.BlockSpec((HID, F_IN), full),
            pl.BlockSpec((HID, 2 * HID), full),
            pl.BlockSpec((HID, 1), full),
            pl.BlockSpec((HID, 1), full),
            pl.BlockSpec((PERIODS, 1), full),
            pl.BlockSpec((PERIODS, HID), full),
            pl.BlockSpec((PERIODS, 1), full),
        ],
        out_specs=pl.BlockSpec((PERIODS, CB), lambda i: (0, i)),
        out_shape=jax.ShapeDtypeStruct((PERIODS, N_PAD), jnp.float32),
    )(ypT, xst, dist, wzT, wlzT, bz, blz, whT, wlhT, bh, blh, attc, woutT,
      boutc)


def kernel(x, edge_index, edge_weight, W_z, b_z, Wl_z, bl_z, W_r, b_r, Wl_r,
           bl_r, W_h, b_h, Wl_h, bl_h, att, W_out, b_out):
    del edge_weight, W_r, b_r, Wl_r, bl_r
    src = edge_index[0]
    dst = edge_index[1]
    pad = jnp.full((E_PAD - E,), N, jnp.int32)
    src4 = jnp.concatenate([src, pad]).reshape(NW, NSG, SG, CHUNK)
    dst4 = jnp.concatenate([dst, pad]).reshape(NW, NSG, SG, CHUNK)
    dst3 = dst4.reshape(NW, CH, CHUNK)

    xflat = x.reshape(N, FT)
    xt = jnp.pad(xflat, ((0, N_PAD - N), (0, 0))).T

    zeros_row = jnp.zeros((N_PAD,), jnp.float32)
    zeros_tab = jnp.zeros((N_PAD, FT), jnp.float32)

    deg2 = _sc_deg(dst3, zeros_row).reshape(NC, N_PAD)
    xst, dist = _prescale(deg2, xt)
    xs = xst.T
    ypart = _sc_scatter(src4, dst4, xs, zeros_tab)
    ypT = jnp.transpose(ypart, (0, 2, 1))

    outT = _gates(
        ypT, xst, dist,
        W_z.T, Wl_z.T, b_z.reshape(HID, 1), bl_z.reshape(HID, 1),
        W_h.T, Wl_h.T, b_h.reshape(HID, 1), bl_h.reshape(HID, 1),
        att.reshape(PERIODS, 1), W_out.T, b_out.reshape(PERIODS, 1),
    )
    return outT.T[:N]


# speedup vs baseline: 366.2279x; 1.1331x over previous
import functools

import jax
import jax.numpy as jnp
from jax import lax
from jax.experimental import pallas as pl
from jax.experimental.pallas import tpu as pltpu
from jax.experimental.pallas import tpu_sc as plsc

N = 50000
E = 800000
F_IN = 2
PERIODS = 12
HID = 32
FT = F_IN * PERIODS

NC = 2
NT = 16
NW = NC * NT
CHUNK = 128
CH = 196
EPT = CH * CHUNK
E_PAD = NW * EPT
ROWS_PER_TILE = 3136
N_PAD = NT * ROWS_PER_TILE

K = 7
SG = 14
NSG = CH // SG
ZR = 784
KD = 14
NSD = CH // KD

CB = 3584
CBG = 1792

_SC_MESH = plsc.VectorSubcoreMesh(core_axis_name="c", subcore_axis_name="s")


@functools.partial(
    pl.kernel,
    out_type=jax.ShapeDtypeStruct((NC * N_PAD,), jnp.float32),
    mesh=_SC_MESH,
    compiler_params=pltpu.CompilerParams(use_tc_tiling_on_sc=False),
    scratch_types=[
        pltpu.VMEM((NSG, SG, CHUNK), jnp.int32),
        pltpu.VMEM((CHUNK,), jnp.float32),
        pltpu.VMEM((ROWS_PER_TILE,), jnp.float32),
        pltpu.VMEM_SHARED((N_PAD,), jnp.float32),
        pltpu.SemaphoreType.DMA,
    ],
)
def _sc_deg(dst_hbm, deg_hbm, idx_v, ones_v, zero_v, deg_sh, sem):
    c = lax.axis_index("c")
    s = lax.axis_index("s")
    w = c * NT + s
    r0 = s * ROWS_PER_TILE
    pltpu.sync_copy(dst_hbm.at[w], idx_v)
    for i in range(CHUNK // 16):
        ones_v[pl.ds(i * 16, 16)] = jnp.full((16,), 1.0, jnp.float32)

    def zfill(i, carry):
        for u in range(16):
            zero_v[pl.ds((i * 16 + u) * 16, 16)] = jnp.zeros((16,), jnp.float32)
        return carry

    lax.fori_loop(0, ROWS_PER_TILE // 256, zfill, 0)
    pltpu.sync_copy(zero_v, deg_sh.at[pl.ds(r0, ROWS_PER_TILE)])
    plsc.subcore_barrier()

    def group(g, carry):
        descs = [
            pltpu.async_copy(ones_v, deg_sh.at[idx_v.at[g, j]], sem, add=True)
            for j in range(KD)
        ]
        for d in descs:
            d.wait()
        return carry

    lax.fori_loop(0, NSD, group, 0)
    plsc.subcore_barrier()
    pltpu.sync_copy(deg_sh.at[pl.ds(r0, ROWS_PER_TILE)], zero_v)
    pltpu.sync_copy(zero_v, deg_hbm.at[pl.ds(c * N_PAD + r0, ROWS_PER_TILE)])


@functools.partial(
    pl.kernel,
    out_type=jax.ShapeDtypeStruct((NC, N_PAD, FT), jnp.float32),
    mesh=_SC_MESH,
    compiler_params=pltpu.CompilerParams(use_tc_tiling_on_sc=False),
    scratch_types=[
        pltpu.VMEM((2, SG, CHUNK), jnp.int32),
        pltpu.VMEM((2, SG, CHUNK), jnp.int32),
        pltpu.VMEM((2, K * CHUNK, FT), jnp.float32),
        pltpu.VMEM_SHARED((N_PAD, FT), jnp.float32),
        pltpu.SemaphoreType.DMA,
        pltpu.SemaphoreType.DMA,
        pltpu.SemaphoreType.DMA,
        pltpu.SemaphoreType.DMA,
    ],
)
def _sc_scatter(src_hbm, dst_hbm, xs_hbm, y_hbm,
                src_v, dst_v, rows_v, y_sh, sem_g0, sem_g1, sem_s0, sem_s1):
    c = lax.axis_index("c")
    s = lax.axis_index("s")
    w = c * NT + s
    r0 = s * ROWS_PER_TILE

    def zfill(i, carry):
        for u in range(8):
            rows_v[0, i * 8 + u, pl.ds(0, 16)] = jnp.zeros((16,), jnp.float32)
            rows_v[0, i * 8 + u, pl.ds(8, 16)] = jnp.zeros((16,), jnp.float32)
        return carry

    lax.fori_loop(0, ZR // 8, zfill, 0)
    zcopies = [
        pltpu.async_copy(rows_v.at[0, pl.ds(0, ZR)],
                         y_sh.at[pl.ds(r0 + q * ZR, ZR)], sem_s0)
        for q in range(ROWS_PER_TILE // ZR)
    ]
    for d in zcopies:
        d.wait()
    plsc.subcore_barrier()

    pltpu.sync_copy(src_hbm.at[w, 0], src_v.at[0])
    pltpu.sync_copy(dst_hbm.at[w, 0], dst_v.at[0])

    def pair(p, carry):
        pb = p % 2
        gets0 = [
            pltpu.async_copy(xs_hbm.at[src_v.at[pb, j]],
                             rows_v.at[0, pl.ds(j * CHUNK, CHUNK)], sem_g0)
            for j in range(K)
        ]
        gets1 = [
            pltpu.async_copy(xs_hbm.at[src_v.at[pb, K + j]],
                             rows_v.at[1, pl.ds(j * CHUNK, CHUNK)], sem_g1)
            for j in range(K)
        ]
        for d in gets0:
            d.wait()
        puts0 = [
            pltpu.async_copy(rows_v.at[0, pl.ds(j * CHUNK, CHUNK)],
                             y_sh.at[dst_v.at[pb, j]], sem_s0, add=True)
            for j in range(K)
        ]
        for d in gets1:
            d.wait()
        puts1 = [
            pltpu.async_copy(rows_v.at[1, pl.ds(j * CHUNK, CHUNK)],
                             y_sh.at[dst_v.at[pb, K + j]], sem_s1, add=True)
            for j in range(K)
        ]

        @pl.when(p + 1 < NSG)
        def _():
            pltpu.sync_copy(src_hbm.at[w, p + 1], src_v.at[(p + 1) % 2])
            pltpu.sync_copy(dst_hbm.at[w, p + 1], dst_v.at[(p + 1) % 2])

        for d in puts0:
            d.wait()
        for d in puts1:
            d.wait()
        return carry

    lax.fori_loop(0, NSG, pair, 0)
    plsc.subcore_barrier()
    for q in range(ROWS_PER_TILE // ZR):
        qr = q * ZR
        pltpu.sync_copy(y_sh.at[pl.ds(r0 + qr, ZR)], rows_v.at[0, pl.ds(0, ZR)])
        pltpu.sync_copy(rows_v.at[0, pl.ds(0, ZR)],
                        y_hbm.at[c, pl.ds(r0 + qr, ZR)])


def _prescale_body(deg_ref, xt_ref, xst_ref, dis_ref):
    dis = lax.rsqrt(deg_ref[0:1, :] + deg_ref[1:2, :] + 1.0)
    dis_ref[...] = dis
    xst_ref[...] = xt_ref[...] * dis


def _prescale(deg2, xt):
    return pl.pallas_call(
        _prescale_body,
        grid=(N_PAD // CB,),
        in_specs=[
            pl.BlockSpec((NC, CB), lambda i: (0, i)),
            pl.BlockSpec((FT, CB), lambda i: (0, i)),
        ],
        out_specs=[
            pl.BlockSpec((FT, CB), lambda i: (0, i)),
            pl.BlockSpec((1, CB), lambda i: (0, i)),
        ],
        out_shape=[
            jax.ShapeDtypeStruct((FT, N_PAD), jnp.float32),
            jax.ShapeDtypeStruct((1, N_PAD), jnp.float32),
        ],
    )(deg2, xt)


def _gates_body(yp_ref, xst_ref, dis_ref, wzT_ref, wlzT_ref, bz_ref, blz_ref,
                whT_ref, wlhT_ref, bh_ref, blh_ref, att_ref, woutT_ref,
                bout_ref, out_ref):
    yf = dis_ref[...] * (yp_ref[0] + yp_ref[1] + xst_ref[...])
    yf1 = jnp.concatenate(
        [yf, jnp.ones((1, yf.shape[1]), jnp.float32)], axis=0)
    wlz1 = wlzT_ref[...][:, :HID]
    wlh1 = wlhT_ref[...][:, :HID]
    azT = jnp.dot(wlz1, wzT_ref[...], preferred_element_type=jnp.float32)
    ahT = jnp.dot(wlh1, whT_ref[...], preferred_element_type=jnp.float32)
    czT = jnp.dot(wlz1, bz_ref[...], preferred_element_type=jnp.float32) + blz_ref[...]
    chT = jnp.dot(wlh1, bh_ref[...], preferred_element_type=jnp.float32) + blh_ref[...]
    a = att_ref[...]
    e = jnp.exp(a - jnp.max(a))
    p = e / jnp.sum(e)
    col = lax.broadcasted_iota(jnp.int32, (HID, FT + 1), 1)
    zero = jnp.zeros((HID, FT + 1), jnp.float32)
    mz = jnp.concatenate(
        [jnp.where(col == t, azT[:, 0:1], zero)
         + jnp.where(col == PERIODS + t, azT[:, 1:2], zero)
         + jnp.where(col == FT, czT, zero) for t in range(PERIODS)], axis=0)
    mh = jnp.concatenate(
        [jnp.where(col == t, ahT[:, 0:1], zero)
         + jnp.where(col == PERIODS + t, ahT[:, 1:2], zero)
         + jnp.where(col == FT, chT, zero) for t in range(PERIODS)], axis=0)
    u = jnp.dot(mz, yf1, preferred_element_type=jnp.float32)
    v = jnp.dot(mh, yf1, preferred_element_type=jnp.float32)
    g = (1.0 - jax.nn.sigmoid(u)) * jnp.tanh(v)
    r32 = lax.broadcasted_iota(jnp.int32, (HID, HID), 0)
    c32 = lax.broadcasted_iota(jnp.int32, (HID, HID), 1)
    eye = jnp.where(r32 == c32, 1.0, 0.0)
    psel = jnp.concatenate([eye * p[t:t + 1, 0:1] for t in range(PERIODS)],
                           axis=1)
    acc = jnp.dot(psel, g, preferred_element_type=jnp.float32)
    out = jnp.dot(woutT_ref[...], jnp.maximum(acc, 0.0),
                  preferred_element_type=jnp.float32)
    out_ref[...] = out + bout_ref[...]


def _gates(ypT, xst, dist, wzT, wlzT, bz, blz, whT, wlhT, bh, blh, attc,
           woutT, boutc):
    full = lambda i: (0, 0)
    return pl.pallas_call(
        _gates_body,
        grid=(N_PAD // CBG,),
        in_specs=[
            pl.BlockSpec((NC, FT, CBG), lambda i: (0, 0, i)),
            pl.BlockSpec((FT, CBG), lambda i: (0, i)),
            pl.BlockSpec((1, CBG), lambda i: (0, i)),
            pl.BlockSpec((HID, F_IN), full),
            pl.BlockSpec((HID, 2 * HID), full),
            pl.BlockSpec((HID, 1), full),
            pl.BlockSpec((HID, 1), full),
            pl.BlockSpec((HID, F_IN), full),
            pl.BlockSpec((HID, 2 * HID), full),
            pl.BlockSpec((HID, 1), full),
            pl.BlockSpec((HID, 1), full),
            pl.BlockSpec((PERIODS, 1), full),
            pl.BlockSpec((PERIODS, HID), full),
            pl.BlockSpec((PERIODS, 1), full),
        ],
        out_specs=pl.BlockSpec((PERIODS, CBG), lambda i: (0, i)),
        out_shape=jax.ShapeDtypeStruct((PERIODS, N_PAD), jnp.float32),
    )(ypT, xst, dist, wzT, wlzT, bz, blz, whT, wlhT, bh, blh, attc, woutT,
      boutc)


def kernel(x, edge_index, edge_weight, W_z, b_z, Wl_z, bl_z, W_r, b_r, Wl_r,
           bl_r, W_h, b_h, Wl_h, bl_h, att, W_out, b_out):
    del edge_weight, W_r, b_r, Wl_r, bl_r
    src = edge_index[0]
    dst = edge_index[1]
    pad = jnp.full((E_PAD - E,), N, jnp.int32)
    src4 = jnp.concatenate([src, pad]).reshape(NW, NSG, SG, CHUNK)
    dst4 = jnp.concatenate([dst, pad]).reshape(NW, NSG, SG, CHUNK)

    xflat = x.reshape(N, FT)
    xt = jnp.pad(xflat, ((0, N_PAD - N), (0, 0))).T

    deg2 = _sc_deg(dst4).reshape(NC, N_PAD)
    xst, dist = _prescale(deg2, xt)
    xs = xst.T
    ypart = _sc_scatter(src4, dst4, xs)
    ypT = jnp.transpose(ypart, (0, 2, 1))

    outT = _gates(
        ypT, xst, dist,
        W_z.T, Wl_z.T, b_z.reshape(HID, 1), bl_z.reshape(HID, 1),
        W_h.T, Wl_h.T, b_h.reshape(HID, 1), bl_h.reshape(HID, 1),
        att.reshape(PERIODS, 1), W_out.T, b_out.reshape(PERIODS, 1),
    )
    return outT.T[:N]


# speedup vs baseline: 399.3115x; 1.0903x over previous
import functools

import jax
import jax.numpy as jnp
from jax import lax
from jax.experimental import pallas as pl
from jax.experimental.pallas import tpu as pltpu
from jax.experimental.pallas import tpu_sc as plsc

N = 50000
E = 800000
F_IN = 2
PERIODS = 12
HID = 32
FT = F_IN * PERIODS

NC = 2
NT = 16
NW = NC * NT
CHUNK = 128
CH = 196
EPT = CH * CHUNK
E_PAD = NW * EPT
ROWS_PER_TILE = 3136
N_PAD = NT * ROWS_PER_TILE

K = 7
SG = 14
NSG = CH // SG
ZR = 784
KD = 14
NSD = CH // KD

CB = 3584
CBG = 1792

_SC_MESH = plsc.VectorSubcoreMesh(core_axis_name="c", subcore_axis_name="s")


@functools.partial(
    pl.kernel,
    out_type=jax.ShapeDtypeStruct((NC * N_PAD,), jnp.float32),
    mesh=_SC_MESH,
    compiler_params=pltpu.CompilerParams(use_tc_tiling_on_sc=False),
    scratch_types=[
        pltpu.VMEM((NSG, SG, CHUNK), jnp.int32),
        pltpu.VMEM((CHUNK,), jnp.float32),
        pltpu.VMEM((ROWS_PER_TILE,), jnp.float32),
        pltpu.VMEM_SHARED((N_PAD,), jnp.float32),
        pltpu.SemaphoreType.DMA,
    ],
)
def _sc_deg(dst_hbm, deg_hbm, idx_v, ones_v, zero_v, deg_sh, sem):
    c = lax.axis_index("c")
    s = lax.axis_index("s")
    w = c * NT + s
    r0 = s * ROWS_PER_TILE
    pltpu.sync_copy(dst_hbm.at[w], idx_v)
    for i in range(CHUNK // 16):
        ones_v[pl.ds(i * 16, 16)] = jnp.full((16,), 1.0, jnp.float32)

    def zfill(i, carry):
        for u in range(16):
            zero_v[pl.ds((i * 16 + u) * 16, 16)] = jnp.zeros((16,), jnp.float32)
        return carry

    lax.fori_loop(0, ROWS_PER_TILE // 256, zfill, 0)
    pltpu.sync_copy(zero_v, deg_sh.at[pl.ds(r0, ROWS_PER_TILE)])
    plsc.subcore_barrier()

    def group(g, carry):
        descs = [
            pltpu.async_copy(ones_v, deg_sh.at[idx_v.at[g, j]], sem, add=True)
            for j in range(KD)
        ]
        for d in descs:
            d.wait()
        return carry

    lax.fori_loop(0, NSD, group, 0)
    plsc.subcore_barrier()
    pltpu.sync_copy(deg_sh.at[pl.ds(r0, ROWS_PER_TILE)], zero_v)
    pltpu.sync_copy(zero_v, deg_hbm.at[pl.ds(c * N_PAD + r0, ROWS_PER_TILE)])


@functools.partial(
    pl.kernel,
    out_type=jax.ShapeDtypeStruct((NC, N_PAD, FT), jnp.float32),
    mesh=_SC_MESH,
    compiler_params=pltpu.CompilerParams(use_tc_tiling_on_sc=False),
    scratch_types=[
        pltpu.VMEM((2, SG, CHUNK), jnp.int32),
        pltpu.VMEM((2, SG, CHUNK), jnp.int32),
        pltpu.VMEM((2, K * CHUNK, FT), jnp.float32),
        pltpu.VMEM_SHARED((N_PAD, FT), jnp.float32),
        pltpu.SemaphoreType.DMA,
        pltpu.SemaphoreType.DMA,
        pltpu.SemaphoreType.DMA,
        pltpu.SemaphoreType.DMA,
    ],
)
def _sc_scatter(src_hbm, dst_hbm, xs_hbm, y_hbm,
                src_v, dst_v, rows_v, y_sh,
                sem_g0, sem_g1, sem_s0, sem_s1):
    c = lax.axis_index("c")
    s = lax.axis_index("s")
    w = c * NT + s
    r0 = s * ROWS_PER_TILE

    def zfill(i, carry):
        for u in range(8):
            rows_v[0, i * 8 + u, pl.ds(0, 16)] = jnp.zeros((16,), jnp.float32)
            rows_v[0, i * 8 + u, pl.ds(8, 16)] = jnp.zeros((16,), jnp.float32)
        return carry

    lax.fori_loop(0, ZR // 8, zfill, 0)
    zcopies = [
        pltpu.async_copy(rows_v.at[0, pl.ds(0, ZR)],
                         y_sh.at[pl.ds(r0 + q * ZR, ZR)], sem_s0)
        for q in range(ROWS_PER_TILE // ZR)
    ]
    for d in zcopies:
        d.wait()
    plsc.subcore_barrier()

    pltpu.sync_copy(src_hbm.at[w, 0], src_v.at[0])
    pltpu.sync_copy(dst_hbm.at[w, 0], dst_v.at[0])

    def pair(p, carry):
        pb = p % 2
        gets0 = [
            pltpu.async_copy(xs_hbm.at[src_v.at[pb, j]],
                             rows_v.at[0, pl.ds(j * CHUNK, CHUNK)], sem_g0)
            for j in range(K)
        ]
        gets1 = [
            pltpu.async_copy(xs_hbm.at[src_v.at[pb, K + j]],
                             rows_v.at[1, pl.ds(j * CHUNK, CHUNK)], sem_g1)
            for j in range(K)
        ]
        for d in gets0:
            d.wait()
        puts0 = [
            pltpu.async_copy(rows_v.at[0, pl.ds(j * CHUNK, CHUNK)],
                             y_sh.at[dst_v.at[pb, j]], sem_s0, add=True)
            for j in range(K)
        ]
        for d in gets1:
            d.wait()
        puts1 = [
            pltpu.async_copy(rows_v.at[1, pl.ds(j * CHUNK, CHUNK)],
                             y_sh.at[dst_v.at[pb, K + j]], sem_s1, add=True)
            for j in range(K)
        ]

        @pl.when(p + 1 < NSG)
        def _():
            pltpu.sync_copy(src_hbm.at[w, p + 1], src_v.at[(p + 1) % 2])
            pltpu.sync_copy(dst_hbm.at[w, p + 1], dst_v.at[(p + 1) % 2])

        for d in puts0:
            d.wait()
        for d in puts1:
            d.wait()
        return carry

    lax.fori_loop(0, NSG, pair, 0)
    plsc.subcore_barrier()
    for q in range(ROWS_PER_TILE // ZR):
        qr = q * ZR
        pltpu.sync_copy(y_sh.at[pl.ds(r0 + qr, ZR)], rows_v.at[0, pl.ds(0, ZR)])
        pltpu.sync_copy(rows_v.at[0, pl.ds(0, ZR)],
                        y_hbm.at[c, pl.ds(r0 + qr, ZR)])


RPW = N_PAD // NW
CHN = 784
NCH = RPW // CHN


@functools.partial(
    pl.kernel,
    out_type=[jax.ShapeDtypeStruct((N_PAD, FT), jnp.float32),
              jax.ShapeDtypeStruct((N_PAD,), jnp.float32)],
    mesh=_SC_MESH,
    compiler_params=pltpu.CompilerParams(use_tc_tiling_on_sc=False,
                                         needs_layout_passes=False),
    scratch_types=[
        pltpu.VMEM((RPW,), jnp.float32),
        pltpu.VMEM((RPW,), jnp.float32),
        pltpu.VMEM((RPW,), jnp.float32),
        pltpu.VMEM((FT, CHN), jnp.float32),
        pltpu.VMEM((CHN, FT), jnp.float32),
        pltpu.SemaphoreType.DMA,
    ],
)
def _sc_scale(deg_hbm, xt_hbm, xs_hbm, dis_hbm,
              deg0_v, deg1_v, dis_v, xtv, xsv, sem):
    c = lax.axis_index("c")
    s = lax.axis_index("s")
    w = c * NT + s
    n0 = w * RPW
    pltpu.sync_copy(deg_hbm.at[pl.ds(n0, RPW)], deg0_v)
    pltpu.sync_copy(deg_hbm.at[pl.ds(N_PAD + n0, RPW)], deg1_v)

    def rsq(i, carry):
        d = deg0_v[pl.ds(i * 16, 16)] + deg1_v[pl.ds(i * 16, 16)] + 1.0
        xi = plsc.bitcast(d, jnp.int32)
        y = plsc.bitcast(jnp.int32(0x5F3759DF) - (xi >> 1), jnp.float32)
        y = y * (1.5 - 0.5 * d * y * y)
        y = y * (1.5 - 0.5 * d * y * y)
        y = y * (1.5 - 0.5 * d * y * y)
        dis_v[pl.ds(i * 16, 16)] = y
        return carry

    lax.fori_loop(0, RPW // 16, rsq, 0)
    pltpu.sync_copy(dis_v, dis_hbm.at[pl.ds(n0, RPW)])
    iota = lax.iota(jnp.int32, 16)
    for ch in range(NCH):
        base = n0 + ch * CHN
        loads = [
            pltpu.async_copy(xt_hbm.at[k, pl.ds(base, CHN)], xtv.at[k], sem)
            for k in range(FT)
        ]
        for d in loads:
            d.wait()

        def tpose(i, carry):
            row = iota + i * 16
            dslice = dis_v[pl.ds(ch * CHN + i * 16, 16)]
            for k in range(FT):
                v = xtv[k, pl.ds(i * 16, 16)] * dslice
                plsc.store_scatter(xsv, [row, jnp.full((16,), k, jnp.int32)], v)
            return carry

        lax.fori_loop(0, CHN // 16, tpose, 0)
        pltpu.sync_copy(xsv, xs_hbm.at[pl.ds(base, CHN)])


def _gates_body(yp_ref, xt_ref, dis_ref, wzT_ref, wlzT_ref, bz_ref, blz_ref,
                whT_ref, wlhT_ref, bh_ref, blh_ref, att_ref, woutT_ref,
                bout_ref, out_ref):
    dis = dis_ref[...]
    yf = dis * (yp_ref[0] + yp_ref[1] + dis * xt_ref[...])
    yf1 = jnp.concatenate(
        [yf, jnp.ones((1, yf.shape[1]), jnp.float32)], axis=0)
    wlz1 = wlzT_ref[...][:, :HID]
    wlh1 = wlhT_ref[...][:, :HID]
    azT = jnp.dot(wlz1, wzT_ref[...], preferred_element_type=jnp.float32)
    ahT = jnp.dot(wlh1, whT_ref[...], preferred_element_type=jnp.float32)
    czT = jnp.dot(wlz1, bz_ref[...], preferred_element_type=jnp.float32) + blz_ref[...]
    chT = jnp.dot(wlh1, bh_ref[...], preferred_element_type=jnp.float32) + blh_ref[...]
    a = att_ref[...]
    e = jnp.exp(a - jnp.max(a))
    p = e / jnp.sum(e)
    col = lax.broadcasted_iota(jnp.int32, (HID, FT + 1), 1)
    zero = jnp.zeros((HID, FT + 1), jnp.float32)
    mz = jnp.concatenate(
        [jnp.where(col == t, azT[:, 0:1], zero)
         + jnp.where(col == PERIODS + t, azT[:, 1:2], zero)
         + jnp.where(col == FT, czT, zero) for t in range(PERIODS)], axis=0)
    mh = jnp.concatenate(
        [jnp.where(col == t, ahT[:, 0:1], zero)
         + jnp.where(col == PERIODS + t, ahT[:, 1:2], zero)
         + jnp.where(col == FT, chT, zero) for t in range(PERIODS)], axis=0)
    u = jnp.dot(mz, yf1, preferred_element_type=jnp.float32)
    v = jnp.dot(mh, yf1, preferred_element_type=jnp.float32)
    g = (1.0 - jax.nn.sigmoid(u)) * jnp.tanh(v)
    r32 = lax.broadcasted_iota(jnp.int32, (HID, HID), 0)
    c32 = lax.broadcasted_iota(jnp.int32, (HID, HID), 1)
    eye = jnp.where(r32 == c32, 1.0, 0.0)
    psel = jnp.concatenate([eye * p[t:t + 1, 0:1] for t in range(PERIODS)],
                           axis=1)
    acc = jnp.dot(psel, g, preferred_element_type=jnp.float32)
    out = jnp.dot(woutT_ref[...], jnp.maximum(acc, 0.0),
                  preferred_element_type=jnp.float32)
    out_ref[...] = out + bout_ref[...]


def _gates(ypT, xt, dist, wzT, wlzT, bz, blz, whT, wlhT, bh, blh, attc,
           woutT, boutc):
    full = lambda i: (0, 0)
    return pl.pallas_call(
        _gates_body,
        grid=(N_PAD // CBG,),
        in_specs=[
            pl.BlockSpec((NC, FT, CBG), lambda i: (0, 0, i)),
            pl.BlockSpec((FT, CBG), lambda i: (0, i)),
            pl.BlockSpec((1, CBG), lambda i: (0, i)),
            pl.BlockSpec((HID, F_IN), full),
            pl.BlockSpec((HID, 2 * HID), full),
            pl.BlockSpec((HID, 1), full),
            pl.BlockSpec((HID, 1), full),
            pl.BlockSpec((HID, F_IN), full),
            pl.BlockSpec((HID, 2 * HID), full),
            pl.BlockSpec((HID, 1), full),
            pl.BlockSpec((HID, 1), full),
            pl.BlockSpec((PERIODS, 1), full),
            pl.BlockSpec((PERIODS, HID), full),
            pl.BlockSpec((PERIODS, 1), full),
        ],
        out_specs=pl.BlockSpec((PERIODS, CBG), lambda i: (0, i)),
        out_shape=jax.ShapeDtypeStruct((PERIODS, N_PAD), jnp.float32),
    )(ypT, xt, dist, wzT, wlzT, bz, blz, whT, wlhT, bh, blh, attc, woutT,
      boutc)


def kernel(x, edge_index, edge_weight, W_z, b_z, Wl_z, bl_z, W_r, b_r, Wl_r,
           bl_r, W_h, b_h, Wl_h, bl_h, att, W_out, b_out):
    del edge_weight, W_r, b_r, Wl_r, bl_r
    src = edge_index[0]
    dst = edge_index[1]
    pad = jnp.full((E_PAD - E,), N, jnp.int32)
    src4 = jnp.concatenate([src, pad]).reshape(NW, NSG, SG, CHUNK)
    dst4 = jnp.concatenate([dst, pad]).reshape(NW, NSG, SG, CHUNK)

    xflat = x.reshape(N, FT)
    xt = jnp.pad(xflat, ((0, N_PAD - N), (0, 0))).T

    deg_flat = _sc_deg(dst4)
    xs, dis = _sc_scale(deg_flat, xt)
    ypart = _sc_scatter(src4, dst4, xs)
    ypT = jnp.transpose(ypart, (0, 2, 1))

    outT = _gates(
        ypT, xt, dis.reshape(1, N_PAD),
        W_z.T, Wl_z.T, b_z.reshape(HID, 1), bl_z.reshape(HID, 1),
        W_h.T, Wl_h.T, b_h.reshape(HID, 1), bl_h.reshape(HID, 1),
        att.reshape(PERIODS, 1), W_out.T, b_out.reshape(PERIODS, 1),
    )
    return outT.T[:N]


# speedup vs baseline: 422.8996x; 1.0591x over previous
import functools

import jax
import jax.numpy as jnp
from jax import lax
from jax.experimental import pallas as pl
from jax.experimental.pallas import tpu as pltpu
from jax.experimental.pallas import tpu_sc as plsc

N = 50000
E = 800000
F_IN = 2
PERIODS = 12
HID = 32
FT = F_IN * PERIODS

NC = 2
NT = 16
NW = NC * NT
CHUNK = 128
CH = 196
EPT = CH * CHUNK
E_PAD = NW * EPT
ROWS_PER_TILE = 3136
N_PAD = NT * ROWS_PER_TILE

K = 7
SG = 14
NSG = CH // SG
ZR = 784
WCH = 112
KD = 14
NSD = CH // KD

CB = 3584
CBG = 1792

_SC_MESH = plsc.VectorSubcoreMesh(core_axis_name="c", subcore_axis_name="s")


@functools.partial(
    pl.kernel,
    out_type=jax.ShapeDtypeStruct((NC * N_PAD,), jnp.float32),
    mesh=_SC_MESH,
    compiler_params=pltpu.CompilerParams(use_tc_tiling_on_sc=False),
    scratch_types=[
        pltpu.VMEM((NSG, SG, CHUNK), jnp.int32),
        pltpu.VMEM((CHUNK,), jnp.float32),
        pltpu.VMEM((ROWS_PER_TILE,), jnp.float32),
        pltpu.VMEM_SHARED((N_PAD,), jnp.float32),
        pltpu.SemaphoreType.DMA,
    ],
)
def _sc_deg(dst_hbm, deg_hbm, idx_v, ones_v, zero_v, deg_sh, sem):
    c = lax.axis_index("c")
    s = lax.axis_index("s")
    w = c * NT + s
    r0 = s * ROWS_PER_TILE
    pltpu.sync_copy(dst_hbm.at[w], idx_v)
    for i in range(CHUNK // 16):
        ones_v[pl.ds(i * 16, 16)] = jnp.full((16,), 1.0, jnp.float32)

    def zfill(i, carry):
        for u in range(16):
            zero_v[pl.ds((i * 16 + u) * 16, 16)] = jnp.zeros((16,), jnp.float32)
        return carry

    lax.fori_loop(0, ROWS_PER_TILE // 256, zfill, 0)
    pltpu.sync_copy(zero_v, deg_sh.at[pl.ds(r0, ROWS_PER_TILE)])
    plsc.subcore_barrier()

    def group(g, carry):
        descs = [
            pltpu.async_copy(ones_v, deg_sh.at[idx_v.at[g, j]], sem, add=True)
            for j in range(KD)
        ]
        for d in descs:
            d.wait()
        return carry

    lax.fori_loop(0, NSD, group, 0)
    plsc.subcore_barrier()
    pltpu.sync_copy(deg_sh.at[pl.ds(r0, ROWS_PER_TILE)], zero_v)
    pltpu.sync_copy(zero_v, deg_hbm.at[pl.ds(c * N_PAD + r0, ROWS_PER_TILE)])


@functools.partial(
    pl.kernel,
    out_type=jax.ShapeDtypeStruct((NC, FT, N_PAD), jnp.float32),
    mesh=_SC_MESH,
    compiler_params=pltpu.CompilerParams(use_tc_tiling_on_sc=False,
                                         needs_layout_passes=False),
    scratch_types=[
        pltpu.VMEM((2, SG, CHUNK), jnp.int32),
        pltpu.VMEM((2, SG, CHUNK), jnp.int32),
        pltpu.VMEM((2, K * CHUNK, FT), jnp.float32),
        pltpu.VMEM((FT, WCH), jnp.float32),
        pltpu.VMEM_SHARED((N_PAD, FT), jnp.float32),
        pltpu.SemaphoreType.DMA,
        pltpu.SemaphoreType.DMA,
        pltpu.SemaphoreType.DMA,
        pltpu.SemaphoreType.DMA,
    ],
)
def _sc_scatter(src_hbm, dst_hbm, xs_hbm, y_hbm,
                src_v, dst_v, rows_v, tgt_v, y_sh,
                sem_g0, sem_g1, sem_s0, sem_s1):
    c = lax.axis_index("c")
    s = lax.axis_index("s")
    w = c * NT + s
    r0 = s * ROWS_PER_TILE

    def zfill(i, carry):
        for u in range(8):
            rows_v[0, i * 8 + u, pl.ds(0, 16)] = jnp.zeros((16,), jnp.float32)
            rows_v[0, i * 8 + u, pl.ds(8, 16)] = jnp.zeros((16,), jnp.float32)
        return carry

    lax.fori_loop(0, ZR // 8, zfill, 0)
    zcopies = [
        pltpu.async_copy(rows_v.at[0, pl.ds(0, ZR)],
                         y_sh.at[pl.ds(r0 + q * ZR, ZR)], sem_s0)
        for q in range(ROWS_PER_TILE // ZR)
    ]
    for d in zcopies:
        d.wait()
    plsc.subcore_barrier()

    pltpu.sync_copy(src_hbm.at[w, 0], src_v.at[0])
    pltpu.sync_copy(dst_hbm.at[w, 0], dst_v.at[0])

    def pair(p, carry):
        pb = p % 2
        gets0 = [
            pltpu.async_copy(xs_hbm.at[src_v.at[pb, j]],
                             rows_v.at[0, pl.ds(j * CHUNK, CHUNK)], sem_g0)
            for j in range(K)
        ]
        gets1 = [
            pltpu.async_copy(xs_hbm.at[src_v.at[pb, K + j]],
                             rows_v.at[1, pl.ds(j * CHUNK, CHUNK)], sem_g1)
            for j in range(K)
        ]
        for d in gets0:
            d.wait()
        puts0 = [
            pltpu.async_copy(rows_v.at[0, pl.ds(j * CHUNK, CHUNK)],
                             y_sh.at[dst_v.at[pb, j]], sem_s0, add=True)
            for j in range(K)
        ]
        for d in gets1:
            d.wait()
        puts1 = [
            pltpu.async_copy(rows_v.at[1, pl.ds(j * CHUNK, CHUNK)],
                             y_sh.at[dst_v.at[pb, K + j]], sem_s1, add=True)
            for j in range(K)
        ]

        @pl.when(p + 1 < NSG)
        def _():
            pltpu.sync_copy(src_hbm.at[w, p + 1], src_v.at[(p + 1) % 2])
            pltpu.sync_copy(dst_hbm.at[w, p + 1], dst_v.at[(p + 1) % 2])

        for d in puts0:
            d.wait()
        for d in puts1:
            d.wait()
        return carry

    lax.fori_loop(0, NSG, pair, 0)
    plsc.subcore_barrier()
    iota = lax.iota(jnp.int32, 16)
    zero16 = jnp.zeros((16,), jnp.int32)

    def wchunk(ch, carry):
        base = r0 + ch * WCH
        pltpu.sync_copy(y_sh.at[pl.ds(base, WCH)], rows_v.at[0, pl.ds(0, WCH)])

        def tpose(i, inner):
            row16 = iota + i * 16
            for k in range(FT):
                v = plsc.load_gather(
                    rows_v, [zero16, row16, jnp.full((16,), k, jnp.int32)])
                tgt_v[k, pl.ds(i * 16, 16)] = v
            return inner

        lax.fori_loop(0, WCH // 16, tpose, 0)
        outs = [
            pltpu.async_copy(tgt_v.at[k], y_hbm.at[c, k, pl.ds(base, WCH)],
                             sem_g1)
            for k in range(FT)
        ]
        for d in outs:
            d.wait()
        return carry

    lax.fori_loop(0, ROWS_PER_TILE // WCH, wchunk, 0)


RPW = N_PAD // NW
CHN = 784
NCH = RPW // CHN


@functools.partial(
    pl.kernel,
    out_type=[jax.ShapeDtypeStruct((N_PAD, FT), jnp.float32),
              jax.ShapeDtypeStruct((N_PAD,), jnp.float32)],
    mesh=_SC_MESH,
    compiler_params=pltpu.CompilerParams(use_tc_tiling_on_sc=False,
                                         needs_layout_passes=False),
    scratch_types=[
        pltpu.VMEM((RPW,), jnp.float32),
        pltpu.VMEM((RPW,), jnp.float32),
        pltpu.VMEM((RPW,), jnp.float32),
        pltpu.VMEM((FT, CHN), jnp.float32),
        pltpu.VMEM((CHN, FT), jnp.float32),
        pltpu.SemaphoreType.DMA,
    ],
)
def _sc_scale(deg_hbm, xt_hbm, xs_hbm, dis_hbm,
              deg0_v, deg1_v, dis_v, xtv, xsv, sem):
    c = lax.axis_index("c")
    s = lax.axis_index("s")
    w = c * NT + s
    n0 = w * RPW
    pltpu.sync_copy(deg_hbm.at[pl.ds(n0, RPW)], deg0_v)
    pltpu.sync_copy(deg_hbm.at[pl.ds(N_PAD + n0, RPW)], deg1_v)

    def rsq(i, carry):
        d = deg0_v[pl.ds(i * 16, 16)] + deg1_v[pl.ds(i * 16, 16)] + 1.0
        xi = plsc.bitcast(d, jnp.int32)
        y = plsc.bitcast(jnp.int32(0x5F3759DF) - (xi >> 1), jnp.float32)
        y = y * (1.5 - 0.5 * d * y * y)
        y = y * (1.5 - 0.5 * d * y * y)
        y = y * (1.5 - 0.5 * d * y * y)
        dis_v[pl.ds(i * 16, 16)] = y
        return carry

    lax.fori_loop(0, RPW // 16, rsq, 0)
    pltpu.sync_copy(dis_v, dis_hbm.at[pl.ds(n0, RPW)])
    iota = lax.iota(jnp.int32, 16)
    for ch in range(NCH):
        base = n0 + ch * CHN
        loads = [
            pltpu.async_copy(xt_hbm.at[k, pl.ds(base, CHN)], xtv.at[k], sem)
            for k in range(FT)
        ]
        for d in loads:
            d.wait()

        def tpose(i, carry):
            row = iota + i * 16
            dslice = dis_v[pl.ds(ch * CHN + i * 16, 16)]
            for k in range(FT):
                v = xtv[k, pl.ds(i * 16, 16)] * dslice
                plsc.store_scatter(xsv, [row, jnp.full((16,), k, jnp.int32)], v)
            return carry

        lax.fori_loop(0, CHN // 16, tpose, 0)
        pltpu.sync_copy(xsv, xs_hbm.at[pl.ds(base, CHN)])


def _gates_body(yp_ref, xt_ref, dis_ref, wzT_ref, wlzT_ref, bz_ref, blz_ref,
                whT_ref, wlhT_ref, bh_ref, blh_ref, att_ref, woutT_ref,
                bout_ref, out_ref):
    dis = dis_ref[...]
    yf = dis * (yp_ref[0] + yp_ref[1] + dis * xt_ref[...])
    yf1 = jnp.concatenate(
        [yf, jnp.ones((1, yf.shape[1]), jnp.float32)], axis=0)
    wlz1 = wlzT_ref[...][:, :HID]
    wlh1 = wlhT_ref[...][:, :HID]
    azT = jnp.dot(wlz1, wzT_ref[...], preferred_element_type=jnp.float32)
    ahT = jnp.dot(wlh1, whT_ref[...], preferred_element_type=jnp.float32)
    czT = jnp.dot(wlz1, bz_ref[...], preferred_element_type=jnp.float32) + blz_ref[...]
    chT = jnp.dot(wlh1, bh_ref[...], preferred_element_type=jnp.float32) + blh_ref[...]
    a = att_ref[...]
    e = jnp.exp(a - jnp.max(a))
    p = e / jnp.sum(e)
    col = lax.broadcasted_iota(jnp.int32, (HID, FT + 1), 1)
    zero = jnp.zeros((HID, FT + 1), jnp.float32)
    mz = jnp.concatenate(
        [jnp.where(col == t, azT[:, 0:1], zero)
         + jnp.where(col == PERIODS + t, azT[:, 1:2], zero)
         + jnp.where(col == FT, czT, zero) for t in range(PERIODS)], axis=0)
    mh = jnp.concatenate(
        [jnp.where(col == t, ahT[:, 0:1], zero)
         + jnp.where(col == PERIODS + t, ahT[:, 1:2], zero)
         + jnp.where(col == FT, chT, zero) for t in range(PERIODS)], axis=0)
    u = jnp.dot(mz, yf1, preferred_element_type=jnp.float32)
    v = jnp.dot(mh, yf1, preferred_element_type=jnp.float32)
    g = (1.0 - jax.nn.sigmoid(u)) * jnp.tanh(v)
    r32 = lax.broadcasted_iota(jnp.int32, (HID, HID), 0)
    c32 = lax.broadcasted_iota(jnp.int32, (HID, HID), 1)
    eye = jnp.where(r32 == c32, 1.0, 0.0)
    psel = jnp.concatenate([eye * p[t:t + 1, 0:1] for t in range(PERIODS)],
                           axis=1)
    acc = jnp.dot(psel, g, preferred_element_type=jnp.float32)
    out = jnp.dot(woutT_ref[...], jnp.maximum(acc, 0.0),
                  preferred_element_type=jnp.float32)
    out_ref[...] = out + bout_ref[...]


def _gates(ypT, xt, dist, wzT, wlzT, bz, blz, whT, wlhT, bh, blh, attc,
           woutT, boutc):
    full = lambda i: (0, 0)
    return pl.pallas_call(
        _gates_body,
        grid=(N_PAD // CBG,),
        in_specs=[
            pl.BlockSpec((NC, FT, CBG), lambda i: (0, 0, i)),
            pl.BlockSpec((FT, CBG), lambda i: (0, i)),
            pl.BlockSpec((1, CBG), lambda i: (0, i)),
            pl.BlockSpec((HID, F_IN), full),
            pl.BlockSpec((HID, 2 * HID), full),
            pl.BlockSpec((HID, 1), full),
            pl.BlockSpec((HID, 1), full),
            pl.BlockSpec((HID, F_IN), full),
            pl.BlockSpec((HID, 2 * HID), full),
            pl.BlockSpec((HID, 1), full),
            pl.BlockSpec((HID, 1), full),
            pl.BlockSpec((PERIODS, 1), full),
            pl.BlockSpec((PERIODS, HID), full),
            pl.BlockSpec((PERIODS, 1), full),
        ],
        out_specs=pl.BlockSpec((PERIODS, CBG), lambda i: (0, i)),
        out_shape=jax.ShapeDtypeStruct((PERIODS, N_PAD), jnp.float32),
    )(ypT, xt, dist, wzT, wlzT, bz, blz, whT, wlhT, bh, blh, attc, woutT,
      boutc)


def kernel(x, edge_index, edge_weight, W_z, b_z, Wl_z, bl_z, W_r, b_r, Wl_r,
           bl_r, W_h, b_h, Wl_h, bl_h, att, W_out, b_out):
    del edge_weight, W_r, b_r, Wl_r, bl_r
    src = edge_index[0]
    dst = edge_index[1]
    pad = jnp.full((E_PAD - E,), N, jnp.int32)
    src4 = jnp.concatenate([src, pad]).reshape(NW, NSG, SG, CHUNK)
    dst4 = jnp.concatenate([dst, pad]).reshape(NW, NSG, SG, CHUNK)

    xflat = x.reshape(N, FT)
    xt = jnp.pad(xflat, ((0, N_PAD - N), (0, 0))).T

    deg_flat = _sc_deg(dst4)
    xs, dis = _sc_scale(deg_flat, xt)
    ypT = _sc_scatter(src4, dst4, xs)

    outT = _gates(
        ypT, xt, dis.reshape(1, N_PAD),
        W_z.T, Wl_z.T, b_z.reshape(HID, 1), bl_z.reshape(HID, 1),
        W_h.T, Wl_h.T, b_h.reshape(HID, 1), bl_h.reshape(HID, 1),
        att.reshape(PERIODS, 1), W_out.T, b_out.reshape(PERIODS, 1),
    )
    return outT.T[:N]


# speedup vs baseline: 426.2850x; 1.0080x over previous
import functools

import jax
import jax.numpy as jnp
from jax import lax
from jax.experimental import pallas as pl
from jax.experimental.pallas import tpu as pltpu
from jax.experimental.pallas import tpu_sc as plsc

N = 50000
E = 800000
F_IN = 2
PERIODS = 12
HID = 32
FT = F_IN * PERIODS

NC = 2
NT = 16
NW = NC * NT
CHUNK = 128
CH = 196
EPT = CH * CHUNK
E_PAD = NW * EPT
ROWS_PER_TILE = 3136
N_PAD = NT * ROWS_PER_TILE

K = 7
SG = 14
NSG = CH // SG
ZR = 784
WCH = 224
KD = 14
NSD = CH // KD

CB = 3584
CBG = 1792

_SC_MESH = plsc.VectorSubcoreMesh(core_axis_name="c", subcore_axis_name="s")


@functools.partial(
    pl.kernel,
    out_type=jax.ShapeDtypeStruct((NC * N_PAD,), jnp.float32),
    mesh=_SC_MESH,
    compiler_params=pltpu.CompilerParams(use_tc_tiling_on_sc=False),
    scratch_types=[
        pltpu.VMEM((NSG, SG, CHUNK), jnp.int32),
        pltpu.VMEM((CHUNK,), jnp.float32),
        pltpu.VMEM((ROWS_PER_TILE,), jnp.float32),
        pltpu.VMEM_SHARED((N_PAD,), jnp.float32),
        pltpu.SemaphoreType.DMA,
    ],
)
def _sc_deg(dst_hbm, deg_hbm, idx_v, ones_v, zero_v, deg_sh, sem):
    c = lax.axis_index("c")
    s = lax.axis_index("s")
    w = c * NT + s
    r0 = s * ROWS_PER_TILE
    pltpu.sync_copy(dst_hbm.at[w], idx_v)
    for i in range(CHUNK // 16):
        ones_v[pl.ds(i * 16, 16)] = jnp.full((16,), 1.0, jnp.float32)

    def zfill(i, carry):
        for u in range(16):
            zero_v[pl.ds((i * 16 + u) * 16, 16)] = jnp.zeros((16,), jnp.float32)
        return carry

    lax.fori_loop(0, ROWS_PER_TILE // 256, zfill, 0)
    pltpu.sync_copy(zero_v, deg_sh.at[pl.ds(r0, ROWS_PER_TILE)])
    plsc.subcore_barrier()

    def group(g, carry):
        descs = [
            pltpu.async_copy(ones_v, deg_sh.at[idx_v.at[g, j]], sem, add=True)
            for j in range(KD)
        ]
        for d in descs:
            d.wait()
        return carry

    lax.fori_loop(0, NSD, group, 0)
    plsc.subcore_barrier()
    pltpu.sync_copy(deg_sh.at[pl.ds(r0, ROWS_PER_TILE)], zero_v)
    pltpu.sync_copy(zero_v, deg_hbm.at[pl.ds(c * N_PAD + r0, ROWS_PER_TILE)])


@functools.partial(
    pl.kernel,
    out_type=jax.ShapeDtypeStruct((NC, FT, N_PAD), jnp.float32),
    mesh=_SC_MESH,
    compiler_params=pltpu.CompilerParams(use_tc_tiling_on_sc=False,
                                         needs_layout_passes=False),
    scratch_types=[
        pltpu.VMEM((2, SG, CHUNK), jnp.int32),
        pltpu.VMEM((2, SG, CHUNK), jnp.int32),
        pltpu.VMEM((2, K * CHUNK, FT), jnp.float32),
        pltpu.VMEM((FT, WCH), jnp.float32),
        pltpu.VMEM_SHARED((N_PAD, FT), jnp.float32),
        pltpu.SemaphoreType.DMA,
        pltpu.SemaphoreType.DMA,
        pltpu.SemaphoreType.DMA,
        pltpu.SemaphoreType.DMA,
    ],
)
def _sc_scatter(src_hbm, dst_hbm, xs_hbm, y_hbm,
                src_v, dst_v, rows_v, tgt_v, y_sh,
                sem_g0, sem_g1, sem_s0, sem_s1):
    c = lax.axis_index("c")
    s = lax.axis_index("s")
    w = c * NT + s
    r0 = s * ROWS_PER_TILE

    def zfill(i, carry):
        for u in range(8):
            rows_v[0, i * 8 + u, pl.ds(0, 16)] = jnp.zeros((16,), jnp.float32)
            rows_v[0, i * 8 + u, pl.ds(8, 16)] = jnp.zeros((16,), jnp.float32)
        return carry

    lax.fori_loop(0, ZR // 8, zfill, 0)
    zcopies = [
        pltpu.async_copy(rows_v.at[0, pl.ds(0, ZR)],
                         y_sh.at[pl.ds(r0 + q * ZR, ZR)], sem_s0)
        for q in range(ROWS_PER_TILE // ZR)
    ]
    for d in zcopies:
        d.wait()
    plsc.subcore_barrier()

    pltpu.sync_copy(src_hbm.at[w, 0], src_v.at[0])
    pltpu.sync_copy(dst_hbm.at[w, 0], dst_v.at[0])

    def pair(p, carry):
        pb = p % 2
        gets0 = [
            pltpu.async_copy(xs_hbm.at[src_v.at[pb, j]],
                             rows_v.at[0, pl.ds(j * CHUNK, CHUNK)], sem_g0)
            for j in range(K)
        ]
        gets1 = [
            pltpu.async_copy(xs_hbm.at[src_v.at[pb, K + j]],
                             rows_v.at[1, pl.ds(j * CHUNK, CHUNK)], sem_g1)
            for j in range(K)
        ]
        for d in gets0:
            d.wait()
        puts0 = [
            pltpu.async_copy(rows_v.at[0, pl.ds(j * CHUNK, CHUNK)],
                             y_sh.at[dst_v.at[pb, j]], sem_s0, add=True)
            for j in range(K)
        ]
        for d in gets1:
            d.wait()
        puts1 = [
            pltpu.async_copy(rows_v.at[1, pl.ds(j * CHUNK, CHUNK)],
                             y_sh.at[dst_v.at[pb, K + j]], sem_s1, add=True)
            for j in range(K)
        ]

        @pl.when(p + 1 < NSG)
        def _():
            pltpu.sync_copy(src_hbm.at[w, p + 1], src_v.at[(p + 1) % 2])
            pltpu.sync_copy(dst_hbm.at[w, p + 1], dst_v.at[(p + 1) % 2])

        for d in puts0:
            d.wait()
        for d in puts1:
            d.wait()
        return carry

    lax.fori_loop(0, NSG, pair, 0)
    plsc.subcore_barrier()
    iota = lax.iota(jnp.int32, 16)
    zero16 = jnp.zeros((16,), jnp.int32)

    def wchunk(ch, carry):
        base = r0 + ch * WCH
        pltpu.sync_copy(y_sh.at[pl.ds(base, WCH)], rows_v.at[0, pl.ds(0, WCH)])

        def tpose(i, inner):
            row16 = iota + i * 16
            for k in range(FT):
                v = plsc.load_gather(
                    rows_v, [zero16, row16, jnp.full((16,), k, jnp.int32)])
                tgt_v[k, pl.ds(i * 16, 16)] = v
            return inner

        lax.fori_loop(0, WCH // 16, tpose, 0)
        outs = [
            pltpu.async_copy(tgt_v.at[k], y_hbm.at[c, k, pl.ds(base, WCH)],
                             sem_g1)
            for k in range(FT)
        ]
        for d in outs:
            d.wait()
        return carry

    lax.fori_loop(0, ROWS_PER_TILE // WCH, wchunk, 0)


RPW = N_PAD // NW
CHN = 784
NCH = RPW // CHN


@functools.partial(
    pl.kernel,
    out_type=[jax.ShapeDtypeStruct((N_PAD, FT), jnp.float32),
              jax.ShapeDtypeStruct((N_PAD,), jnp.float32)],
    mesh=_SC_MESH,
    compiler_params=pltpu.CompilerParams(use_tc_tiling_on_sc=False,
                                         needs_layout_passes=False),
    scratch_types=[
        pltpu.VMEM((RPW,), jnp.float32),
        pltpu.VMEM((RPW,), jnp.float32),
        pltpu.VMEM((RPW,), jnp.float32),
        pltpu.VMEM((FT, CHN), jnp.float32),
        pltpu.VMEM((CHN, FT), jnp.float32),
        pltpu.SemaphoreType.DMA,
    ],
)
def _sc_scale(deg_hbm, xt_hbm, xs_hbm, dis_hbm,
              deg0_v, deg1_v, dis_v, xtv, xsv, sem):
    c = lax.axis_index("c")
    s = lax.axis_index("s")
    w = c * NT + s
    n0 = w * RPW
    pltpu.sync_copy(deg_hbm.at[pl.ds(n0, RPW)], deg0_v)
    pltpu.sync_copy(deg_hbm.at[pl.ds(N_PAD + n0, RPW)], deg1_v)

    def rsq(i, carry):
        d = deg0_v[pl.ds(i * 16, 16)] + deg1_v[pl.ds(i * 16, 16)] + 1.0
        xi = plsc.bitcast(d, jnp.int32)
        y = plsc.bitcast(jnp.int32(0x5F3759DF) - (xi >> 1), jnp.float32)
        y = y * (1.5 - 0.5 * d * y * y)
        y = y * (1.5 - 0.5 * d * y * y)
        y = y * (1.5 - 0.5 * d * y * y)
        dis_v[pl.ds(i * 16, 16)] = y
        return carry

    lax.fori_loop(0, RPW // 16, rsq, 0)
    pltpu.sync_copy(dis_v, dis_hbm.at[pl.ds(n0, RPW)])
    iota = lax.iota(jnp.int32, 16)
    for ch in range(NCH):
        base = n0 + ch * CHN
        loads = [
            pltpu.async_copy(xt_hbm.at[k, pl.ds(base, CHN)], xtv.at[k], sem)
            for k in range(FT)
        ]
        for d in loads:
            d.wait()

        def tpose(i, carry):
            row = iota + i * 16
            dslice = dis_v[pl.ds(ch * CHN + i * 16, 16)]
            for k in range(FT):
                v = xtv[k, pl.ds(i * 16, 16)] * dslice
                plsc.store_scatter(xsv, [row, jnp.full((16,), k, jnp.int32)], v)
            return carry

        lax.fori_loop(0, CHN // 16, tpose, 0)
        pltpu.sync_copy(xsv, xs_hbm.at[pl.ds(base, CHN)])


def _gates_body(yp_ref, xt_ref, dis_ref, wzT_ref, wlzT_ref, bz_ref, blz_ref,
                whT_ref, wlhT_ref, bh_ref, blh_ref, att_ref, woutT_ref,
                bout_ref, out_ref):
    dis = dis_ref[...]
    yf = dis * (yp_ref[0] + yp_ref[1] + dis * xt_ref[...])
    yf1 = jnp.concatenate(
        [yf, jnp.ones((1, yf.shape[1]), jnp.float32)], axis=0)
    wlz1 = wlzT_ref[...][:, :HID]
    wlh1 = wlhT_ref[...][:, :HID]
    azT = jnp.dot(wlz1, wzT_ref[...], preferred_element_type=jnp.float32)
    ahT = jnp.dot(wlh1, whT_ref[...], preferred_element_type=jnp.float32)
    czT = jnp.dot(wlz1, bz_ref[...], preferred_element_type=jnp.float32) + blz_ref[...]
    chT = jnp.dot(wlh1, bh_ref[...], preferred_element_type=jnp.float32) + blh_ref[...]
    a = att_ref[...]
    e = jnp.exp(a - jnp.max(a))
    p = e / jnp.sum(e)
    col = lax.broadcasted_iota(jnp.int32, (HID, FT + 1), 1)
    zero = jnp.zeros((HID, FT + 1), jnp.float32)
    mz = jnp.concatenate(
        [jnp.where(col == t, azT[:, 0:1], zero)
         + jnp.where(col == PERIODS + t, azT[:, 1:2], zero)
         + jnp.where(col == FT, czT, zero) for t in range(PERIODS)], axis=0)
    mh = jnp.concatenate(
        [jnp.where(col == t, ahT[:, 0:1], zero)
         + jnp.where(col == PERIODS + t, ahT[:, 1:2], zero)
         + jnp.where(col == FT, chT, zero) for t in range(PERIODS)], axis=0)
    u = jnp.dot(mz, yf1, preferred_element_type=jnp.float32)
    v = jnp.dot(mh, yf1, preferred_element_type=jnp.float32)
    g = (1.0 - jax.nn.sigmoid(u)) * jnp.tanh(v)
    r32 = lax.broadcasted_iota(jnp.int32, (HID, HID), 0)
    c32 = lax.broadcasted_iota(jnp.int32, (HID, HID), 1)
    eye = jnp.where(r32 == c32, 1.0, 0.0)
    psel = jnp.concatenate([eye * p[t:t + 1, 0:1] for t in range(PERIODS)],
                           axis=1)
    acc = jnp.dot(psel, g, preferred_element_type=jnp.float32)
    out = jnp.dot(woutT_ref[...], jnp.maximum(acc, 0.0),
                  preferred_element_type=jnp.float32)
    out_ref[...] = out + bout_ref[...]


def _gates(ypT, xt, dist, wzT, wlzT, bz, blz, whT, wlhT, bh, blh, attc,
           woutT, boutc):
    full = lambda i: (0, 0)
    return pl.pallas_call(
        _gates_body,
        grid=(N_PAD // CBG,),
        in_specs=[
            pl.BlockSpec((NC, FT, CBG), lambda i: (0, 0, i)),
            pl.BlockSpec((FT, CBG), lambda i: (0, i)),
            pl.BlockSpec((1, CBG), lambda i: (0, i)),
            pl.BlockSpec((HID, F_IN), full),
            pl.BlockSpec((HID, 2 * HID), full),
            pl.BlockSpec((HID, 1), full),
            pl.BlockSpec((HID, 1), full),
            pl.BlockSpec((HID, F_IN), full),
            pl.BlockSpec((HID, 2 * HID), full),
            pl.BlockSpec((HID, 1), full),
            pl.BlockSpec((HID, 1), full),
            pl.BlockSpec((PERIODS, 1), full),
            pl.BlockSpec((PERIODS, HID), full),
            pl.BlockSpec((PERIODS, 1), full),
        ],
        out_specs=pl.BlockSpec((PERIODS, CBG), lambda i: (0, i)),
        out_shape=jax.ShapeDtypeStruct((PERIODS, N_PAD), jnp.float32),
    )(ypT, xt, dist, wzT, wlzT, bz, blz, whT, wlhT, bh, blh, attc, woutT,
      boutc)


def kernel(x, edge_index, edge_weight, W_z, b_z, Wl_z, bl_z, W_r, b_r, Wl_r,
           bl_r, W_h, b_h, Wl_h, bl_h, att, W_out, b_out):
    del edge_weight, W_r, b_r, Wl_r, bl_r
    src = edge_index[0]
    dst = edge_index[1]
    pad = jnp.full((E_PAD - E,), N, jnp.int32)
    src4 = jnp.concatenate([src, pad]).reshape(NW, NSG, SG, CHUNK)
    dst4 = jnp.concatenate([dst, pad]).reshape(NW, NSG, SG, CHUNK)

    xflat = x.reshape(N, FT)
    xt = jnp.pad(xflat, ((0, N_PAD - N), (0, 0))).T

    deg_flat = _sc_deg(dst4)
    xs, dis = _sc_scale(deg_flat, xt)
    ypT = _sc_scatter(src4, dst4, xs)

    outT = _gates(
        ypT, xt, dis.reshape(1, N_PAD),
        W_z.T, Wl_z.T, b_z.reshape(HID, 1), bl_z.reshape(HID, 1),
        W_h.T, Wl_h.T, b_h.reshape(HID, 1), bl_h.reshape(HID, 1),
        att.reshape(PERIODS, 1), W_out.T, b_out.reshape(PERIODS, 1),
    )
    return outT.T[:N]
